# Initial kernel scaffold; baseline (speedup 1.0000x reference)
import jax
import jax.numpy as jnp
from jax.experimental import pallas as pl


def kernel(x, edge_index, edge_attr, edge_index3, edge_attr3, gx, bx, Wn, bn_, We, be, W1, c1b, Wih1, Whh1, bih1, bhh1, g1, b1, Wl1, bl1, Wl2, bl2, W2, c2b, Wih2, Whh2, bih2, bhh2, Wlw, Wlb, gN, bN):
    raise NotImplementedError("write your pallas kernel here")


# speedup vs baseline: 1.4853x; 1.4853x over previous
import functools

import jax
import jax.numpy as jnp
from jax import lax
from jax.experimental import pallas as pl
from jax.experimental.pallas import tpu as pltpu
from jax.experimental.pallas import tpu_sc as plsc

_SLOPE = (1.0 / 8.0 + 1.0 / 3.0) / 2.0
_EPS = 1e-5

_NC = 2
_NS = 16
_NW = _NC * _NS
_CH = 128


def _rrelu(v):
    return jnp.where(v >= 0, v, _SLOPE * v)


def _bn_scale(g):
    return g * (1.0 / jnp.sqrt(1.0 + _EPS))


def _sc_mesh():
    return plsc.VectorSubcoreMesh(core_axis_name="c", subcore_axis_name="s")


_SC_PARAMS = pltpu.CompilerParams(use_tc_tiling_on_sc=False)


def _build_gather(D, P):
    chunks = P // (_NW * _CH)
    per_w = chunks * _CH

    def body(table, idx, out, idx_v, rows_v):
        wid = lax.axis_index("s") * _NC + lax.axis_index("c")

        def step(j, c):
            base = wid * per_w + j * _CH
            pltpu.sync_copy(idx.at[pl.ds(base, _CH)], idx_v)
            pltpu.sync_copy(table.at[idx_v], rows_v)
            pltpu.sync_copy(rows_v, out.at[pl.ds(base, _CH)])
            return c

        lax.fori_loop(0, chunks, step, 0)

    return pl.kernel(
        body,
        out_type=jax.ShapeDtypeStruct((P, D), jnp.float32),
        mesh=_sc_mesh(),
        scratch_types=[
            pltpu.VMEM((_CH,), jnp.int32),
            pltpu.VMEM((_CH, D), jnp.float32),
        ],
        compiler_params=_SC_PARAMS,
    )


def _build_scatter(D, P, nacc):
    chunks = P // (_NW * _CH)
    per_w = chunks * _CH
    rpt = nacc // _NS

    def body(rows, idx, zeros, out, idx_v, rows_v, acc_sh):
        cid = lax.axis_index("c")
        sid = lax.axis_index("s")
        wid = sid * _NC + cid
        pltpu.sync_copy(zeros.at[pl.ds(sid * rpt, rpt)],
                        acc_sh.at[pl.ds(sid * rpt, rpt)])
        plsc.subcore_barrier()

        def step(j, c):
            base = wid * per_w + j * _CH
            pltpu.sync_copy(idx.at[pl.ds(base, _CH)], idx_v)
            pltpu.sync_copy(rows.at[pl.ds(base, _CH)], rows_v)
            pltpu.sync_copy(rows_v, acc_sh.at[idx_v], add=True)
            return c

        lax.fori_loop(0, chunks, step, 0)
        plsc.subcore_barrier()
        pltpu.sync_copy(acc_sh.at[pl.ds(sid * rpt, rpt)],
                        out.at[cid, pl.ds(sid * rpt, rpt)])

    return pl.kernel(
        body,
        out_type=jax.ShapeDtypeStruct((_NC, nacc, D), jnp.float32),
        mesh=_sc_mesh(),
        scratch_types=[
            pltpu.VMEM((_CH,), jnp.int32),
            pltpu.VMEM((_CH, D), jnp.float32),
            pltpu.VMEM_SHARED((nacc, D), jnp.float32),
        ],
        compiler_params=_SC_PARAMS,
    )


def _build_count(P, nacc):
    chunks = P // (_NW * _CH)
    per_w = chunks * _CH
    rpt = nacc // _NS
    D = 16

    def body(idx, ones, zeros, out, idx_v, ones_v, acc_sh):
        cid = lax.axis_index("c")
        sid = lax.axis_index("s")
        wid = sid * _NC + cid
        pltpu.sync_copy(zeros.at[pl.ds(sid * rpt, rpt)],
                        acc_sh.at[pl.ds(sid * rpt, rpt)])
        pltpu.sync_copy(ones, ones_v)
        plsc.subcore_barrier()

        def step(j, c):
            base = wid * per_w + j * _CH
            pltpu.sync_copy(idx.at[pl.ds(base, _CH)], idx_v)
            pltpu.sync_copy(ones_v, acc_sh.at[idx_v], add=True)
            return c

        lax.fori_loop(0, chunks, step, 0)
        plsc.subcore_barrier()
        pltpu.sync_copy(acc_sh.at[pl.ds(sid * rpt, rpt)],
                        out.at[cid, pl.ds(sid * rpt, rpt)])

    return pl.kernel(
        body,
        out_type=jax.ShapeDtypeStruct((_NC, nacc, D), jnp.float32),
        mesh=_sc_mesh(),
        scratch_types=[
            pltpu.VMEM((_CH,), jnp.int32),
            pltpu.VMEM((_CH, D), jnp.float32),
            pltpu.VMEM_SHARED((nacc, D), jnp.float32),
        ],
        compiler_params=_SC_PARAMS,
    )


def _full(shape):
    return pl.BlockSpec(shape, lambda *_: tuple(0 for _ in shape))


def _node_prologue(x, gx2, bx2, WnT, bn2, nb):
    n, fin = x.shape
    dout = WnT.shape[1]

    def body(x_ref, g_ref, b_ref, w_ref, bn_ref, o_ref):
        xb = x_ref[...] * (g_ref[...] * (1.0 / jnp.sqrt(1.0 + _EPS))) + b_ref[...]
        o_ref[...] = _rrelu(
            jnp.dot(xb, w_ref[...], preferred_element_type=jnp.float32)
            + bn_ref[...])

    return pl.pallas_call(
        body,
        grid=(n // nb,),
        in_specs=[
            pl.BlockSpec((nb, fin), lambda i: (i, 0)),
            _full(gx2.shape), _full(bx2.shape), _full(WnT.shape), _full(bn2.shape),
        ],
        out_specs=pl.BlockSpec((nb, dout), lambda i: (i, 0)),
        out_shape=jax.ShapeDtypeStruct((n, dout), jnp.float32),
    )(x, gx2, bx2, WnT, bn2)


def _edge_prologue(eattr, WeT, be2, eb):
    p, fin = eattr.shape
    dout = WeT.shape[1]

    def body(e_ref, w_ref, b_ref, o_ref):
        o_ref[...] = _rrelu(
            jnp.dot(e_ref[...], w_ref[...], preferred_element_type=jnp.float32)
            + b_ref[...])

    return pl.pallas_call(
        body,
        grid=(p // eb,),
        in_specs=[
            pl.BlockSpec((eb, fin), lambda i: (i, 0)),
            _full(WeT.shape), _full(be2.shape),
        ],
        out_specs=pl.BlockSpec((eb, dout), lambda i: (i, 0)),
        out_shape=jax.ShapeDtypeStruct((p, dout), jnp.float32),
    )(eattr, WeT, be2)


def _edge_conv(xj, ea, Vall, K, dout, eb):
    p, din = xj.shape

    def body(x_ref, e_ref, v_ref, o_ref):
        proj = jnp.dot(x_ref[...], v_ref[...],
                       preferred_element_type=jnp.float32)
        acc = e_ref[:, 0:1] * proj[:, 0:dout]
        for k in range(1, K):
            acc = acc + e_ref[:, k:k + 1] * proj[:, k * dout:(k + 1) * dout]
        o_ref[...] = acc

    return pl.pallas_call(
        body,
        grid=(p // eb,),
        in_specs=[
            pl.BlockSpec((eb, din), lambda i: (i, 0)),
            pl.BlockSpec((eb, ea.shape[1]), lambda i: (i, 0)),
            _full(Vall.shape),
        ],
        out_specs=pl.BlockSpec((eb, dout), lambda i: (i, 0)),
        out_shape=jax.ShapeDtypeStruct((p, dout), jnp.float32),
    )(xj, ea, Vall)


def _gru_update(accp, cntp, h, cb2, WihT, WhhT, bih2, bhh2, nb):
    n, dim = h.shape
    nacc = accp.shape[1]

    def body(a_ref, c_ref, h_ref, cb_ref, wi_ref, wh_ref, bi_ref, bh_ref,
             o_ref):
        acc = a_ref[0] + a_ref[1]
        cnt = c_ref[0][:, 0:1] + c_ref[1][:, 0:1]
        m = _rrelu(acc / jnp.maximum(cnt, 1.0) + cb_ref[...])
        hv = h_ref[...]
        gi = jnp.dot(m, wi_ref[...], preferred_element_type=jnp.float32) \
            + bi_ref[...]
        gh = jnp.dot(hv, wh_ref[...], preferred_element_type=jnp.float32) \
            + bh_ref[...]
        r = jax.nn.sigmoid(gi[:, :dim] + gh[:, :dim])
        z = jax.nn.sigmoid(gi[:, dim:2 * dim] + gh[:, dim:2 * dim])
        nn = jnp.tanh(gi[:, 2 * dim:] + r * gh[:, 2 * dim:])
        o_ref[...] = (1.0 - z) * nn + z * hv

    return pl.pallas_call(
        body,
        grid=(n // nb,),
        in_specs=[
            pl.BlockSpec((_NC, nb, dim), lambda i: (0, i, 0)),
            pl.BlockSpec((_NC, nb, 16), lambda i: (0, i, 0)),
            pl.BlockSpec((nb, dim), lambda i: (i, 0)),
            _full(cb2.shape), _full(WihT.shape), _full(WhhT.shape),
            _full(bih2.shape), _full(bhh2.shape),
        ],
        out_specs=pl.BlockSpec((nb, dim), lambda i: (i, 0)),
        out_shape=jax.ShapeDtypeStruct((n, dim), jnp.float32),
    )(accp, cntp, h, cb2, WihT, WhhT, bih2, bhh2)


def _gru_update_mlp(accp, cntp, h, cb2, WihT, WhhT, bih2, bhh2,
                    g12, b12, Wl1T, bl12, Wl2T, bl22, nb):
    n, dim = h.shape
    dmid = Wl1T.shape[1]

    def body(a_ref, c_ref, h_ref, cb_ref, wi_ref, wh_ref, bi_ref, bh_ref,
             g_ref, b_ref, w1_ref, b1_ref, w2_ref, b2_ref, o_ref):
        acc = a_ref[0] + a_ref[1]
        cnt = c_ref[0][:, 0:1] + c_ref[1][:, 0:1]
        m = _rrelu(acc / jnp.maximum(cnt, 1.0) + cb_ref[...])
        hv = h_ref[...]
        gi = jnp.dot(m, wi_ref[...], preferred_element_type=jnp.float32) \
            + bi_ref[...]
        gh = jnp.dot(hv, wh_ref[...], preferred_element_type=jnp.float32) \
            + bh_ref[...]
        r = jax.nn.sigmoid(gi[:, :dim] + gh[:, :dim])
        z = jax.nn.sigmoid(gi[:, dim:2 * dim] + gh[:, dim:2 * dim])
        nn = jnp.tanh(gi[:, 2 * dim:] + r * gh[:, 2 * dim:])
        h2 = (1.0 - z) * nn + z * hv
        xb = h2 * (g_ref[...] * (1.0 / jnp.sqrt(1.0 + _EPS))) + b_ref[...]
        t = _rrelu(jnp.dot(xb, w1_ref[...],
                           preferred_element_type=jnp.float32) + b1_ref[...])
        o_ref[...] = _rrelu(jnp.dot(t, w2_ref[...],
                                    preferred_element_type=jnp.float32)
                            + b2_ref[...])

    return pl.pallas_call(
        body,
        grid=(n // nb,),
        in_specs=[
            pl.BlockSpec((_NC, nb, dim), lambda i: (0, i, 0)),
            pl.BlockSpec((_NC, nb, 16), lambda i: (0, i, 0)),
            pl.BlockSpec((nb, dim), lambda i: (i, 0)),
            _full(cb2.shape), _full(WihT.shape), _full(WhhT.shape),
            _full(bih2.shape), _full(bhh2.shape),
            _full(g12.shape), _full(b12.shape), _full(Wl1T.shape),
            _full(bl12.shape), _full(Wl2T.shape), _full(bl22.shape),
        ],
        out_specs=pl.BlockSpec((nb, dmid), lambda i: (i, 0)),
        out_shape=jax.ShapeDtypeStruct((n, dmid), jnp.float32),
    )(accp, cntp, h, cb2, WihT, WhhT, bih2, bhh2,
      g12, b12, Wl1T, bl12, Wl2T, bl22)


def _readout(rows, ea3, gN2, bN2, WlwT, Wlb2, e3, eb):
    d2 = rows.shape[1]

    def body(t0_ref, t1_ref, e_ref, g_ref, b_ref, ww_ref, wb_ref, o_ref):
        t0 = t0_ref[...]
        t1 = t1_ref[...]
        yhat = jnp.concatenate(
            [0.5 * (t0 + t1), t0 * t1, (t0 - t1) ** 2], axis=1)
        yhat = yhat * (g_ref[...] * (1.0 / jnp.sqrt(1.0 + _EPS))) + b_ref[...]
        ev = e_ref[...]
        w = jnp.dot(ev, ww_ref[...], preferred_element_type=jnp.float32)
        b = jnp.sum(ev * wb_ref[...], axis=1, keepdims=True)
        o_ref[...] = jnp.sum(yhat * w, axis=1, keepdims=True) + b

    nblk = e3 // eb
    return pl.pallas_call(
        body,
        grid=(nblk,),
        in_specs=[
            pl.BlockSpec((eb, d2), lambda i: (i, 0)),
            pl.BlockSpec((eb, d2), lambda i: (i + nblk, 0)),
            pl.BlockSpec((eb, ea3.shape[1]), lambda i: (i, 0)),
            _full(gN2.shape), _full(bN2.shape), _full(WlwT.shape),
            _full(Wlb2.shape),
        ],
        out_specs=pl.BlockSpec((eb, 1), lambda i: (i, 0)),
        out_shape=jax.ShapeDtypeStruct((e3, 1), jnp.float32),
    )(rows, rows, ea3, gN2, bN2, WlwT, Wlb2)


def _pad_to(n, mult):
    return ((n + mult - 1) // mult) * mult


def kernel(x, edge_index, edge_attr, edge_index3, edge_attr3, gx, bx, Wn,
           bn_, We, be, W1, c1b, Wih1, Whh1, bih1, bhh1, g1, b1, Wl1, bl1,
           Wl2, bl2, W2, c2b, Wih2, Whh2, bih2, bhh2, Wlw, Wlb, gN, bN):
    f32 = jnp.float32
    N = x.shape[0]
    E = edge_index.shape[1]
    E3 = edge_index3.shape[1]
    dim = Wn.shape[0]
    d2 = 2 * dim
    K1 = We.shape[0]
    K2 = edge_attr3.shape[1]
    NB = 2000
    EB = _NW * _CH

    P1 = _pad_to(E, EB)
    P2 = _pad_to(2 * E3, EB)
    nacc = _pad_to(N + 1, NB)

    ei = edge_index.astype(jnp.int32)
    ei3 = edge_index3.astype(jnp.int32)
    src1 = jnp.pad(ei[0], (0, P1 - E))
    dst1 = jnp.pad(ei[1], (0, P1 - E), constant_values=N)
    src2 = jnp.pad(jnp.concatenate([ei3[0], ei3[1]]), (0, P2 - 2 * E3))
    dst2 = jnp.pad(jnp.concatenate([ei3[1], ei3[0]]), (0, P2 - 2 * E3),
                   constant_values=N)
    eattr_p = jnp.pad(edge_attr, ((0, P1 - E), (0, 0)))
    ea3f = jnp.pad(jnp.concatenate([edge_attr3, edge_attr3], axis=0),
                   ((0, P2 - 2 * E3), (0, 0)))

    zeros1 = jnp.zeros((nacc, dim), f32)
    zeros2 = jnp.zeros((nacc, d2), f32)
    zerosc = jnp.zeros((nacc, 16), f32)
    ones = jnp.ones((_CH, 16), f32)

    WnT = Wn.T
    WeT = jnp.pad(We.T, ((0, 0), (0, 16 - K1)))
    be2 = jnp.pad(be, (0, 16 - K1)).reshape(1, 16)
    Vall1 = W1.reshape(dim, dim, K1).transpose(0, 2, 1).reshape(dim, K1 * dim)
    Vall2 = W2.reshape(d2, d2, K2).transpose(0, 2, 1).reshape(d2, K2 * d2)
    gx2 = gx.reshape(1, -1)
    bx2 = bx.reshape(1, -1)
    bn2 = bn_.reshape(1, -1)
    c1b2 = c1b.reshape(1, -1)
    c2b2 = c2b.reshape(1, -1)
    Wih1T = Wih1.T
    Whh1T = Whh1.T
    bih1r = bih1.reshape(1, -1)
    bhh1r = bhh1.reshape(1, -1)
    Wih2T = Wih2.T
    Whh2T = Whh2.T
    bih2r = bih2.reshape(1, -1)
    bhh2r = bhh2.reshape(1, -1)
    g12 = g1.reshape(1, -1)
    b12 = b1.reshape(1, -1)
    Wl1T = Wl1.T
    bl12 = bl1.reshape(1, -1)
    Wl2T = Wl2.T
    bl22 = bl2.reshape(1, -1)
    gN2 = gN.reshape(1, -1)
    bN2 = bN.reshape(1, -1)
    WlwT = Wlw.T
    Wlb2 = Wlb.reshape(1, -1)

    gather1 = _build_gather(dim, P1)
    gather2 = _build_gather(d2, P2)
    scatter1 = _build_scatter(dim, P1, nacc)
    scatter2 = _build_scatter(d2, P2, nacc)
    count1 = _build_count(P1, nacc)
    count2 = _build_count(P2, nacc)

    out0 = _node_prologue(x, gx2, bx2, WnT, bn2, NB)
    ea = _edge_prologue(eattr_p, WeT, be2, EB)
    cnt1 = count1(dst1, ones, zerosc)
    cnt2 = count2(dst2, ones, zerosc)

    h = out0
    xj = gather1(h, src1)
    m = _edge_conv(xj, ea, Vall1, K1, dim, EB)
    accp = scatter1(m, dst1, zeros1)
    h = _gru_update(accp, cnt1, h, c1b2, Wih1T, Whh1T, bih1r, bhh1r, NB)

    xj = gather1(h, src1)
    m = _edge_conv(xj, ea, Vall1, K1, dim, EB)
    accp = scatter1(m, dst1, zeros1)
    h = _gru_update_mlp(accp, cnt1, h, c1b2, Wih1T, Whh1T, bih1r, bhh1r,
                        g12, b12, Wl1T, bl12, Wl2T, bl22, NB)

    for _ in range(2):
        xj = gather2(h, src2)
        m = _edge_conv(xj, ea3f, Vall2, K2, d2, EB)
        accp = scatter2(m, dst2, zeros2)
        h = _gru_update(accp, cnt2, h, c2b2, Wih2T, Whh2T, bih2r, bhh2r, NB)

    rows = gather2(h, src2)
    res = _readout(rows, edge_attr3, gN2, bN2, WlwT, Wlb2, E3, NB)
    return res[:, 0]


# speedup vs baseline: 1.6136x; 1.0864x over previous
import functools

import jax
import jax.numpy as jnp
from jax import lax
from jax.experimental import pallas as pl
from jax.experimental.pallas import tpu as pltpu
from jax.experimental.pallas import tpu_sc as plsc

_SLOPE = (1.0 / 8.0 + 1.0 / 3.0) / 2.0
_EPS = 1e-5

_NC = 2
_NS = 16
_NW = _NC * _NS
_CH = 128


def _rrelu(v):
    return jnp.where(v >= 0, v, _SLOPE * v)


def _bn_scale(g):
    return g * (1.0 / jnp.sqrt(1.0 + _EPS))


def _sc_mesh():
    return plsc.VectorSubcoreMesh(core_axis_name="c", subcore_axis_name="s")


_SC_PARAMS = pltpu.CompilerParams(use_tc_tiling_on_sc=False)


def _build_gather(D, P):
    chunks = P // (_NW * _CH)
    per_w = chunks * _CH

    def body(table, idx3, out, idx_v, rows_v, gsem):
        wid = lax.axis_index("s") * _NC + lax.axis_index("c")
        pltpu.sync_copy(idx3.at[wid], idx_v)

        def fire(j, c):
            pltpu.async_copy(table.at[idx_v.at[j]],
                             rows_v.at[pl.ds(j * _CH, _CH)], gsem)
            return c

        lax.fori_loop(0, chunks, fire, 0)

        def drain(j, c):
            pltpu.make_async_copy(table.at[idx_v.at[0]],
                                  rows_v.at[pl.ds(0, _CH)], gsem).wait()
            return c

        lax.fori_loop(0, chunks, drain, 0)
        pltpu.sync_copy(rows_v, out.at[pl.ds(wid * per_w, per_w)])

    return pl.kernel(
        body,
        out_type=jax.ShapeDtypeStruct((P, D), jnp.float32),
        mesh=_sc_mesh(),
        scratch_types=[
            pltpu.VMEM((chunks, _CH), jnp.int32),
            pltpu.VMEM((per_w, D), jnp.float32),
            pltpu.SemaphoreType.DMA,
        ],
        compiler_params=_SC_PARAMS,
    )


def _build_scatter(D, P, nacc):
    chunks = P // (_NW * _CH)
    per_w = chunks * _CH
    rpt = nacc // _NS

    def body(rows, idx3, zeros, out, idx_v, rows_v, acc_sh, lsem, asem):
        cid = lax.axis_index("c")
        sid = lax.axis_index("s")
        wid = sid * _NC + cid
        pltpu.async_copy(idx3.at[wid], idx_v, lsem)
        pltpu.async_copy(rows.at[pl.ds(wid * per_w, per_w)], rows_v, lsem)
        pltpu.sync_copy(zeros.at[pl.ds(sid * rpt, rpt)],
                        acc_sh.at[pl.ds(sid * rpt, rpt)])
        plsc.subcore_barrier()
        pltpu.make_async_copy(idx3.at[wid], idx_v, lsem).wait()
        pltpu.make_async_copy(rows.at[pl.ds(wid * per_w, per_w)], rows_v,
                              lsem).wait()

        def fire(j, c):
            pltpu.async_copy(rows_v.at[pl.ds(j * _CH, _CH)],
                             acc_sh.at[idx_v.at[j]], asem, add=True)
            return c

        lax.fori_loop(0, chunks, fire, 0)

        def drain(j, c):
            pltpu.make_async_copy(rows_v.at[pl.ds(0, _CH)],
                                  acc_sh.at[idx_v.at[0]], asem).wait()
            return c

        lax.fori_loop(0, chunks, drain, 0)
        plsc.subcore_barrier()
        pltpu.sync_copy(acc_sh.at[pl.ds(sid * rpt, rpt)],
                        out.at[cid, pl.ds(sid * rpt, rpt)])

    return pl.kernel(
        body,
        out_type=jax.ShapeDtypeStruct((_NC, nacc, D), jnp.float32),
        mesh=_sc_mesh(),
        scratch_types=[
            pltpu.VMEM((chunks, _CH), jnp.int32),
            pltpu.VMEM((per_w, D), jnp.float32),
            pltpu.VMEM_SHARED((nacc, D), jnp.float32),
            pltpu.SemaphoreType.DMA,
            pltpu.SemaphoreType.DMA,
        ],
        compiler_params=_SC_PARAMS,
    )


def _build_counts(P1, P2, nacc):
    chunks1 = P1 // (_NW * _CH)
    chunks2 = P2 // (_NW * _CH)
    rpt = nacc // _NS
    D = 16

    def body(idx3a, idx3b, ones, zeros, out1, out2,
             idxa_v, idxb_v, ones_v, acca_sh, accb_sh, lsem, asem):
        cid = lax.axis_index("c")
        sid = lax.axis_index("s")
        wid = sid * _NC + cid
        pltpu.async_copy(idx3a.at[wid], idxa_v, lsem)
        pltpu.async_copy(idx3b.at[wid], idxb_v, lsem)
        pltpu.sync_copy(ones, ones_v)
        pltpu.sync_copy(zeros.at[pl.ds(sid * rpt, rpt)],
                        acca_sh.at[pl.ds(sid * rpt, rpt)])
        pltpu.sync_copy(zeros.at[pl.ds(sid * rpt, rpt)],
                        accb_sh.at[pl.ds(sid * rpt, rpt)])
        plsc.subcore_barrier()
        pltpu.make_async_copy(idx3a.at[wid], idxa_v, lsem).wait()
        pltpu.make_async_copy(idx3b.at[wid], idxb_v, lsem).wait()

        def fire_a(j, c):
            pltpu.async_copy(ones_v, acca_sh.at[idxa_v.at[j]], asem,
                             add=True)
            return c

        def fire_b(j, c):
            pltpu.async_copy(ones_v, accb_sh.at[idxb_v.at[j]], asem,
                             add=True)
            return c

        lax.fori_loop(0, chunks1, fire_a, 0)
        lax.fori_loop(0, chunks2, fire_b, 0)

        def drain(j, c):
            pltpu.make_async_copy(ones_v, acca_sh.at[idxa_v.at[0]],
                                  asem).wait()
            return c

        lax.fori_loop(0, chunks1 + chunks2, drain, 0)
        plsc.subcore_barrier()
        pltpu.sync_copy(acca_sh.at[pl.ds(sid * rpt, rpt)],
                        out1.at[cid, pl.ds(sid * rpt, rpt)])
        pltpu.sync_copy(accb_sh.at[pl.ds(sid * rpt, rpt)],
                        out2.at[cid, pl.ds(sid * rpt, rpt)])

    return pl.kernel(
        body,
        out_type=[jax.ShapeDtypeStruct((_NC, nacc, D), jnp.float32),
                  jax.ShapeDtypeStruct((_NC, nacc, D), jnp.float32)],
        mesh=_sc_mesh(),
        scratch_types=[
            pltpu.VMEM((chunks1, _CH), jnp.int32),
            pltpu.VMEM((chunks2, _CH), jnp.int32),
            pltpu.VMEM((_CH, D), jnp.float32),
            pltpu.VMEM_SHARED((nacc, D), jnp.float32),
            pltpu.VMEM_SHARED((nacc, D), jnp.float32),
            pltpu.SemaphoreType.DMA,
            pltpu.SemaphoreType.DMA,
        ],
        compiler_params=_SC_PARAMS,
    )


def _full(shape):
    return pl.BlockSpec(shape, lambda *_: tuple(0 for _ in shape))


def _node_prologue(x, gx2, bx2, WnT, bn2, nb):
    n, fin = x.shape
    dout = WnT.shape[1]

    def body(x_ref, g_ref, b_ref, w_ref, bn_ref, o_ref):
        xb = x_ref[...] * (g_ref[...] * (1.0 / jnp.sqrt(1.0 + _EPS))) + b_ref[...]
        o_ref[...] = _rrelu(
            jnp.dot(xb, w_ref[...], preferred_element_type=jnp.float32)
            + bn_ref[...])

    return pl.pallas_call(
        body,
        grid=(n // nb,),
        in_specs=[
            pl.BlockSpec((nb, fin), lambda i: (i, 0)),
            _full(gx2.shape), _full(bx2.shape), _full(WnT.shape), _full(bn2.shape),
        ],
        out_specs=pl.BlockSpec((nb, dout), lambda i: (i, 0)),
        out_shape=jax.ShapeDtypeStruct((n, dout), jnp.float32),
    )(x, gx2, bx2, WnT, bn2)


def _edge_prologue(eattr, WeT, be2, eb):
    p, fin = eattr.shape
    dout = WeT.shape[1]

    def body(e_ref, w_ref, b_ref, o_ref):
        o_ref[...] = _rrelu(
            jnp.dot(e_ref[...], w_ref[...], preferred_element_type=jnp.float32)
            + b_ref[...])

    return pl.pallas_call(
        body,
        grid=(p // eb,),
        in_specs=[
            pl.BlockSpec((eb, fin), lambda i: (i, 0)),
            _full(WeT.shape), _full(be2.shape),
        ],
        out_specs=pl.BlockSpec((eb, dout), lambda i: (i, 0)),
        out_shape=jax.ShapeDtypeStruct((p, dout), jnp.float32),
    )(eattr, WeT, be2)


def _edge_conv(xj, ea, Vall, K, dout, eb):
    p, din = xj.shape

    def body(x_ref, e_ref, v_ref, o_ref):
        proj = jnp.dot(x_ref[...], v_ref[...],
                       preferred_element_type=jnp.float32)
        acc = e_ref[:, 0:1] * proj[:, 0:dout]
        for k in range(1, K):
            acc = acc + e_ref[:, k:k + 1] * proj[:, k * dout:(k + 1) * dout]
        o_ref[...] = acc

    return pl.pallas_call(
        body,
        grid=(p // eb,),
        in_specs=[
            pl.BlockSpec((eb, din), lambda i: (i, 0)),
            pl.BlockSpec((eb, ea.shape[1]), lambda i: (i, 0)),
            _full(Vall.shape),
        ],
        out_specs=pl.BlockSpec((eb, dout), lambda i: (i, 0)),
        out_shape=jax.ShapeDtypeStruct((p, dout), jnp.float32),
    )(xj, ea, Vall)


def _gru_update(accp, cntp, h, cb2, WihT, WhhT, bih2, bhh2, nb):
    n, dim = h.shape
    nacc = accp.shape[1]

    def body(a_ref, c_ref, h_ref, cb_ref, wi_ref, wh_ref, bi_ref, bh_ref,
             o_ref):
        acc = a_ref[0] + a_ref[1]
        cnt = c_ref[0][:, 0:1] + c_ref[1][:, 0:1]
        m = _rrelu(acc / jnp.maximum(cnt, 1.0) + cb_ref[...])
        hv = h_ref[...]
        gi = jnp.dot(m, wi_ref[...], preferred_element_type=jnp.float32) \
            + bi_ref[...]
        gh = jnp.dot(hv, wh_ref[...], preferred_element_type=jnp.float32) \
            + bh_ref[...]
        r = jax.nn.sigmoid(gi[:, :dim] + gh[:, :dim])
        z = jax.nn.sigmoid(gi[:, dim:2 * dim] + gh[:, dim:2 * dim])
        nn = jnp.tanh(gi[:, 2 * dim:] + r * gh[:, 2 * dim:])
        o_ref[...] = (1.0 - z) * nn + z * hv

    return pl.pallas_call(
        body,
        grid=(n // nb,),
        in_specs=[
            pl.BlockSpec((_NC, nb, dim), lambda i: (0, i, 0)),
            pl.BlockSpec((_NC, nb, 16), lambda i: (0, i, 0)),
            pl.BlockSpec((nb, dim), lambda i: (i, 0)),
            _full(cb2.shape), _full(WihT.shape), _full(WhhT.shape),
            _full(bih2.shape), _full(bhh2.shape),
        ],
        out_specs=pl.BlockSpec((nb, dim), lambda i: (i, 0)),
        out_shape=jax.ShapeDtypeStruct((n, dim), jnp.float32),
    )(accp, cntp, h, cb2, WihT, WhhT, bih2, bhh2)


def _gru_update_mlp(accp, cntp, h, cb2, WihT, WhhT, bih2, bhh2,
                    g12, b12, Wl1T, bl12, Wl2T, bl22, nb):
    n, dim = h.shape
    dmid = Wl1T.shape[1]

    def body(a_ref, c_ref, h_ref, cb_ref, wi_ref, wh_ref, bi_ref, bh_ref,
             g_ref, b_ref, w1_ref, b1_ref, w2_ref, b2_ref, o_ref):
        acc = a_ref[0] + a_ref[1]
        cnt = c_ref[0][:, 0:1] + c_ref[1][:, 0:1]
        m = _rrelu(acc / jnp.maximum(cnt, 1.0) + cb_ref[...])
        hv = h_ref[...]
        gi = jnp.dot(m, wi_ref[...], preferred_element_type=jnp.float32) \
            + bi_ref[...]
        gh = jnp.dot(hv, wh_ref[...], preferred_element_type=jnp.float32) \
            + bh_ref[...]
        r = jax.nn.sigmoid(gi[:, :dim] + gh[:, :dim])
        z = jax.nn.sigmoid(gi[:, dim:2 * dim] + gh[:, dim:2 * dim])
        nn = jnp.tanh(gi[:, 2 * dim:] + r * gh[:, 2 * dim:])
        h2 = (1.0 - z) * nn + z * hv
        xb = h2 * (g_ref[...] * (1.0 / jnp.sqrt(1.0 + _EPS))) + b_ref[...]
        t = _rrelu(jnp.dot(xb, w1_ref[...],
                           preferred_element_type=jnp.float32) + b1_ref[...])
        o_ref[...] = _rrelu(jnp.dot(t, w2_ref[...],
                                    preferred_element_type=jnp.float32)
                            + b2_ref[...])

    return pl.pallas_call(
        body,
        grid=(n // nb,),
        in_specs=[
            pl.BlockSpec((_NC, nb, dim), lambda i: (0, i, 0)),
            pl.BlockSpec((_NC, nb, 16), lambda i: (0, i, 0)),
            pl.BlockSpec((nb, dim), lambda i: (i, 0)),
            _full(cb2.shape), _full(WihT.shape), _full(WhhT.shape),
            _full(bih2.shape), _full(bhh2.shape),
            _full(g12.shape), _full(b12.shape), _full(Wl1T.shape),
            _full(bl12.shape), _full(Wl2T.shape), _full(bl22.shape),
        ],
        out_specs=pl.BlockSpec((nb, dmid), lambda i: (i, 0)),
        out_shape=jax.ShapeDtypeStruct((n, dmid), jnp.float32),
    )(accp, cntp, h, cb2, WihT, WhhT, bih2, bhh2,
      g12, b12, Wl1T, bl12, Wl2T, bl22)


def _readout(rows, ea3, gN2, bN2, WlwT, Wlb2, e3, eb):
    d2 = rows.shape[1]

    def body(t0_ref, t1_ref, e_ref, g_ref, b_ref, ww_ref, wb_ref, o_ref):
        t0 = t0_ref[...]
        t1 = t1_ref[...]
        yhat = jnp.concatenate(
            [0.5 * (t0 + t1), t0 * t1, (t0 - t1) ** 2], axis=1)
        yhat = yhat * (g_ref[...] * (1.0 / jnp.sqrt(1.0 + _EPS))) + b_ref[...]
        ev = e_ref[...]
        w = jnp.dot(ev, ww_ref[...], preferred_element_type=jnp.float32)
        b = jnp.sum(ev * wb_ref[...], axis=1, keepdims=True)
        o_ref[...] = jnp.sum(yhat * w, axis=1, keepdims=True) + b

    nblk = e3 // eb
    return pl.pallas_call(
        body,
        grid=(nblk,),
        in_specs=[
            pl.BlockSpec((eb, d2), lambda i: (i, 0)),
            pl.BlockSpec((eb, d2), lambda i: (i + nblk, 0)),
            pl.BlockSpec((eb, ea3.shape[1]), lambda i: (i, 0)),
            _full(gN2.shape), _full(bN2.shape), _full(WlwT.shape),
            _full(Wlb2.shape),
        ],
        out_specs=pl.BlockSpec((eb, 1), lambda i: (i, 0)),
        out_shape=jax.ShapeDtypeStruct((e3, 1), jnp.float32),
    )(rows, rows, ea3, gN2, bN2, WlwT, Wlb2)


def _pad_to(n, mult):
    return ((n + mult - 1) // mult) * mult


def kernel(x, edge_index, edge_attr, edge_index3, edge_attr3, gx, bx, Wn,
           bn_, We, be, W1, c1b, Wih1, Whh1, bih1, bhh1, g1, b1, Wl1, bl1,
           Wl2, bl2, W2, c2b, Wih2, Whh2, bih2, bhh2, Wlw, Wlb, gN, bN):
    f32 = jnp.float32
    N = x.shape[0]
    E = edge_index.shape[1]
    E3 = edge_index3.shape[1]
    dim = Wn.shape[0]
    d2 = 2 * dim
    K1 = We.shape[0]
    K2 = edge_attr3.shape[1]
    NB = 2000
    EB = _NW * _CH

    P1 = _pad_to(E, EB)
    P2 = _pad_to(2 * E3, EB)
    nacc = _pad_to(N + 1, NB)

    ei = edge_index.astype(jnp.int32)
    ei3 = edge_index3.astype(jnp.int32)
    ch1 = P1 // (_NW * _CH)
    ch2 = P2 // (_NW * _CH)
    src1 = jnp.pad(ei[0], (0, P1 - E)).reshape(_NW, ch1, _CH)
    dst1 = jnp.pad(ei[1], (0, P1 - E),
                   constant_values=N).reshape(_NW, ch1, _CH)
    src2 = jnp.pad(jnp.concatenate([ei3[0], ei3[1]]),
                   (0, P2 - 2 * E3)).reshape(_NW, ch2, _CH)
    dst2 = jnp.pad(jnp.concatenate([ei3[1], ei3[0]]), (0, P2 - 2 * E3),
                   constant_values=N).reshape(_NW, ch2, _CH)
    eattr_p = jnp.pad(edge_attr, ((0, P1 - E), (0, 0)))
    ea3f = jnp.pad(jnp.concatenate([edge_attr3, edge_attr3], axis=0),
                   ((0, P2 - 2 * E3), (0, 0)))

    zeros1 = jnp.zeros((nacc, dim), f32)
    zeros2 = jnp.zeros((nacc, d2), f32)
    zerosc = jnp.zeros((nacc, 16), f32)
    ones = jnp.ones((_CH, 16), f32)

    WnT = Wn.T
    WeT = jnp.pad(We.T, ((0, 0), (0, 16 - K1)))
    be2 = jnp.pad(be, (0, 16 - K1)).reshape(1, 16)
    Vall1 = W1.reshape(dim, dim, K1).transpose(0, 2, 1).reshape(dim, K1 * dim)
    Vall2 = W2.reshape(d2, d2, K2).transpose(0, 2, 1).reshape(d2, K2 * d2)
    gx2 = gx.reshape(1, -1)
    bx2 = bx.reshape(1, -1)
    bn2 = bn_.reshape(1, -1)
    c1b2 = c1b.reshape(1, -1)
    c2b2 = c2b.reshape(1, -1)
    Wih1T = Wih1.T
    Whh1T = Whh1.T
    bih1r = bih1.reshape(1, -1)
    bhh1r = bhh1.reshape(1, -1)
    Wih2T = Wih2.T
    Whh2T = Whh2.T
    bih2r = bih2.reshape(1, -1)
    bhh2r = bhh2.reshape(1, -1)
    g12 = g1.reshape(1, -1)
    b12 = b1.reshape(1, -1)
    Wl1T = Wl1.T
    bl12 = bl1.reshape(1, -1)
    Wl2T = Wl2.T
    bl22 = bl2.reshape(1, -1)
    gN2 = gN.reshape(1, -1)
    bN2 = bN.reshape(1, -1)
    WlwT = Wlw.T
    Wlb2 = Wlb.reshape(1, -1)

    gather1 = _build_gather(dim, P1)
    gather2 = _build_gather(d2, P2)
    scatter1 = _build_scatter(dim, P1, nacc)
    scatter2 = _build_scatter(d2, P2, nacc)
    counts = _build_counts(P1, P2, nacc)

    out0 = _node_prologue(x, gx2, bx2, WnT, bn2, NB)
    ea = _edge_prologue(eattr_p, WeT, be2, EB)
    cnt1, cnt2 = counts(dst1, dst2, ones, zerosc)

    h = out0
    xj = gather1(h, src1)
    m = _edge_conv(xj, ea, Vall1, K1, dim, EB)
    accp = scatter1(m, dst1, zeros1)
    h = _gru_update(accp, cnt1, h, c1b2, Wih1T, Whh1T, bih1r, bhh1r, NB)

    xj = gather1(h, src1)
    m = _edge_conv(xj, ea, Vall1, K1, dim, EB)
    accp = scatter1(m, dst1, zeros1)
    h = _gru_update_mlp(accp, cnt1, h, c1b2, Wih1T, Whh1T, bih1r, bhh1r,
                        g12, b12, Wl1T, bl12, Wl2T, bl22, NB)

    for _ in range(2):
        xj = gather2(h, src2)
        m = _edge_conv(xj, ea3f, Vall2, K2, d2, EB)
        accp = scatter2(m, dst2, zeros2)
        h = _gru_update(accp, cnt2, h, c2b2, Wih2T, Whh2T, bih2r, bhh2r, NB)

    rows = gather2(h, src2)
    res = _readout(rows, edge_attr3, gN2, bN2, WlwT, Wlb2, E3, NB)
    return res[:, 0]


# speedup vs baseline: 3.2195x; 1.9952x over previous
import functools

import jax
import jax.numpy as jnp
from jax import lax
from jax.experimental import pallas as pl
from jax.experimental.pallas import tpu as pltpu
from jax.experimental.pallas import tpu_sc as plsc

_SLOPE = (1.0 / 8.0 + 1.0 / 3.0) / 2.0
_EPS = 1e-5

_NC = 2
_NS = 16
_NW = _NC * _NS
_CH = 128


def _rrelu(v):
    return jnp.where(v >= 0, v, _SLOPE * v)


def _bn_scale(g):
    return g * (1.0 / jnp.sqrt(1.0 + _EPS))


def _sc_mesh():
    return plsc.VectorSubcoreMesh(core_axis_name="c", subcore_axis_name="s")


_SC_PARAMS = pltpu.CompilerParams(use_tc_tiling_on_sc=False)


def _build_gather(D, P):
    chunks = P // (_NW * _CH)
    per_w = chunks * _CH

    def body(table, idx3, out, idx_v, rows_v, gsem):
        wid = lax.axis_index("s") * _NC + lax.axis_index("c")
        pltpu.sync_copy(idx3.at[wid], idx_v)

        def fire(j, c):
            pltpu.async_copy(table.at[idx_v.at[j]],
                             rows_v.at[pl.ds(j * _CH, _CH)], gsem)
            return c

        lax.fori_loop(0, chunks, fire, 0)

        def drain(j, c):
            pltpu.make_async_copy(table.at[idx_v.at[0]],
                                  rows_v.at[pl.ds(0, _CH)], gsem).wait()
            return c

        lax.fori_loop(0, chunks, drain, 0)
        pltpu.sync_copy(rows_v, out.at[pl.ds(wid * per_w, per_w)])

    return pl.kernel(
        body,
        out_type=jax.ShapeDtypeStruct((P, D), jnp.float32),
        mesh=_sc_mesh(),
        scratch_types=[
            pltpu.VMEM((chunks, _CH), jnp.int32),
            pltpu.VMEM((per_w, D), jnp.float32),
            pltpu.SemaphoreType.DMA,
        ],
        compiler_params=_SC_PARAMS,
    )


def _build_scatter(D, P, nacc):
    chunks = P // (_NW * _CH)
    per_w = chunks * _CH
    rpt = nacc // _NS

    def body(rows, idx3, zeros, out, idx_v, rows_v, acc_sh, lsem, asem):
        cid = lax.axis_index("c")
        sid = lax.axis_index("s")
        wid = sid * _NC + cid
        pltpu.async_copy(idx3.at[wid], idx_v, lsem)
        pltpu.async_copy(rows.at[pl.ds(wid * per_w, per_w)], rows_v, lsem)
        pltpu.sync_copy(zeros.at[pl.ds(sid * rpt, rpt)],
                        acc_sh.at[pl.ds(sid * rpt, rpt)])
        plsc.subcore_barrier()
        pltpu.make_async_copy(idx3.at[wid], idx_v, lsem).wait()
        pltpu.make_async_copy(rows.at[pl.ds(wid * per_w, per_w)], rows_v,
                              lsem).wait()

        def fire(j, c):
            pltpu.async_copy(rows_v.at[pl.ds(j * _CH, _CH)],
                             acc_sh.at[idx_v.at[j]], asem, add=True)
            return c

        lax.fori_loop(0, chunks, fire, 0)

        def drain(j, c):
            pltpu.make_async_copy(rows_v.at[pl.ds(0, _CH)],
                                  acc_sh.at[idx_v.at[0]], asem).wait()
            return c

        lax.fori_loop(0, chunks, drain, 0)
        plsc.subcore_barrier()
        pltpu.sync_copy(acc_sh.at[pl.ds(sid * rpt, rpt)],
                        out.at[cid, pl.ds(sid * rpt, rpt)])

    return pl.kernel(
        body,
        out_type=jax.ShapeDtypeStruct((_NC, nacc, D), jnp.float32),
        mesh=_sc_mesh(),
        scratch_types=[
            pltpu.VMEM((chunks, _CH), jnp.int32),
            pltpu.VMEM((per_w, D), jnp.float32),
            pltpu.VMEM_SHARED((nacc, D), jnp.float32),
            pltpu.SemaphoreType.DMA,
            pltpu.SemaphoreType.DMA,
        ],
        compiler_params=_SC_PARAMS,
    )


def _build_counts(P1, P2, nacc):
    chunks1 = P1 // (_NW * _CH)
    chunks2 = P2 // (_NW * _CH)
    rpt = nacc // _NS
    D = 16

    def body(idx3a, idx3b, ones, zeros, out1, out2,
             idxa_v, idxb_v, ones_v, acca_sh, accb_sh, lsem, asem):
        cid = lax.axis_index("c")
        sid = lax.axis_index("s")
        wid = sid * _NC + cid
        pltpu.async_copy(idx3a.at[wid], idxa_v, lsem)
        pltpu.async_copy(idx3b.at[wid], idxb_v, lsem)
        pltpu.sync_copy(ones, ones_v)
        pltpu.sync_copy(zeros.at[pl.ds(sid * rpt, rpt)],
                        acca_sh.at[pl.ds(sid * rpt, rpt)])
        pltpu.sync_copy(zeros.at[pl.ds(sid * rpt, rpt)],
                        accb_sh.at[pl.ds(sid * rpt, rpt)])
        plsc.subcore_barrier()
        pltpu.make_async_copy(idx3a.at[wid], idxa_v, lsem).wait()
        pltpu.make_async_copy(idx3b.at[wid], idxb_v, lsem).wait()

        def fire_a(j, c):
            pltpu.async_copy(ones_v, acca_sh.at[idxa_v.at[j]], asem,
                             add=True)
            return c

        def fire_b(j, c):
            pltpu.async_copy(ones_v, accb_sh.at[idxb_v.at[j]], asem,
                             add=True)
            return c

        lax.fori_loop(0, chunks1, fire_a, 0)
        lax.fori_loop(0, chunks2, fire_b, 0)

        def drain(j, c):
            pltpu.make_async_copy(ones_v, acca_sh.at[idxa_v.at[0]],
                                  asem).wait()
            return c

        lax.fori_loop(0, chunks1 + chunks2, drain, 0)
        plsc.subcore_barrier()
        pltpu.sync_copy(acca_sh.at[pl.ds(sid * rpt, rpt)],
                        out1.at[cid, pl.ds(sid * rpt, rpt)])
        pltpu.sync_copy(accb_sh.at[pl.ds(sid * rpt, rpt)],
                        out2.at[cid, pl.ds(sid * rpt, rpt)])

    return pl.kernel(
        body,
        out_type=[jax.ShapeDtypeStruct((_NC, nacc, D), jnp.float32),
                  jax.ShapeDtypeStruct((_NC, nacc, D), jnp.float32)],
        mesh=_sc_mesh(),
        scratch_types=[
            pltpu.VMEM((chunks1, _CH), jnp.int32),
            pltpu.VMEM((chunks2, _CH), jnp.int32),
            pltpu.VMEM((_CH, D), jnp.float32),
            pltpu.VMEM_SHARED((nacc, D), jnp.float32),
            pltpu.VMEM_SHARED((nacc, D), jnp.float32),
            pltpu.SemaphoreType.DMA,
            pltpu.SemaphoreType.DMA,
        ],
        compiler_params=_SC_PARAMS,
    )


def _full(shape):
    return pl.BlockSpec(shape, lambda *_: tuple(0 for _ in shape))


def _node_prologue(x, gx2, bx2, WnT, bn2, nb):
    n, fin = x.shape
    dout = WnT.shape[1]

    def body(x_ref, g_ref, b_ref, w_ref, bn_ref, o_ref):
        xb = x_ref[...] * (g_ref[...] * (1.0 / jnp.sqrt(1.0 + _EPS))) + b_ref[...]
        o_ref[...] = _rrelu(
            jnp.dot(xb, w_ref[...], preferred_element_type=jnp.float32)
            + bn_ref[...])

    return pl.pallas_call(
        body,
        grid=(n // nb,),
        in_specs=[
            pl.BlockSpec((nb, fin), lambda i: (i, 0)),
            _full(gx2.shape), _full(bx2.shape), _full(WnT.shape), _full(bn2.shape),
        ],
        out_specs=pl.BlockSpec((nb, dout), lambda i: (i, 0)),
        out_shape=jax.ShapeDtypeStruct((n, dout), jnp.float32),
    )(x, gx2, bx2, WnT, bn2)


def _edge_conv1(xj, eattr, WeT, be2, Bexp, Vall, C, dout, eb):
    p, din = xj.shape

    def body(x_ref, e_ref, w_ref, b_ref, bx_ref, v_ref, c_ref, o_ref):
        ea = _rrelu(jnp.dot(e_ref[...], w_ref[...],
                            preferred_element_type=jnp.float32) + b_ref[...])
        eaexp = jnp.dot(ea, bx_ref[...], preferred_element_type=jnp.float32)
        proj = jnp.dot(x_ref[...], v_ref[...],
                       preferred_element_type=jnp.float32)
        o_ref[...] = jnp.dot(eaexp * proj, c_ref[...],
                             preferred_element_type=jnp.float32)

    return pl.pallas_call(
        body,
        grid=(p // eb,),
        in_specs=[
            pl.BlockSpec((eb, din), lambda i: (i, 0)),
            pl.BlockSpec((eb, eattr.shape[1]), lambda i: (i, 0)),
            _full(WeT.shape), _full(be2.shape), _full(Bexp.shape),
            _full(Vall.shape), _full(C.shape),
        ],
        out_specs=pl.BlockSpec((eb, dout), lambda i: (i, 0)),
        out_shape=jax.ShapeDtypeStruct((p, dout), jnp.float32),
    )(xj, eattr, WeT, be2, Bexp, Vall, C)


def _edge_conv2(xj, ea, Bexp, Vall, C, dout, eb):
    p, din = xj.shape

    def body(x_ref, e_ref, bx_ref, v_ref, c_ref, o_ref):
        eaexp = jnp.dot(e_ref[...], bx_ref[...],
                        preferred_element_type=jnp.float32)
        proj = jnp.dot(x_ref[...], v_ref[...],
                       preferred_element_type=jnp.float32)
        o_ref[...] = jnp.dot(eaexp * proj, c_ref[...],
                             preferred_element_type=jnp.float32)

    return pl.pallas_call(
        body,
        grid=(p // eb,),
        in_specs=[
            pl.BlockSpec((eb, din), lambda i: (i, 0)),
            pl.BlockSpec((eb, ea.shape[1]), lambda i: (i, 0)),
            _full(Bexp.shape), _full(Vall.shape), _full(C.shape),
        ],
        out_specs=pl.BlockSpec((eb, dout), lambda i: (i, 0)),
        out_shape=jax.ShapeDtypeStruct((p, dout), jnp.float32),
    )(xj, ea, Bexp, Vall, C)


def _gru_update(accp, cntp, h, cb2, WihT, WhhT, bih2, bhh2, nb):
    n, dim = h.shape
    nacc = accp.shape[1]

    def body(a_ref, c_ref, h_ref, cb_ref, wi_ref, wh_ref, bi_ref, bh_ref,
             o_ref):
        acc = a_ref[0] + a_ref[1]
        cnt = c_ref[0][:, 0:1] + c_ref[1][:, 0:1]
        m = _rrelu(acc / jnp.maximum(cnt, 1.0) + cb_ref[...])
        hv = h_ref[...]
        gi = jnp.dot(m, wi_ref[...], preferred_element_type=jnp.float32) \
            + bi_ref[...]
        gh = jnp.dot(hv, wh_ref[...], preferred_element_type=jnp.float32) \
            + bh_ref[...]
        r = jax.nn.sigmoid(gi[:, :dim] + gh[:, :dim])
        z = jax.nn.sigmoid(gi[:, dim:2 * dim] + gh[:, dim:2 * dim])
        nn = jnp.tanh(gi[:, 2 * dim:] + r * gh[:, 2 * dim:])
        o_ref[...] = (1.0 - z) * nn + z * hv

    return pl.pallas_call(
        body,
        grid=(n // nb,),
        in_specs=[
            pl.BlockSpec((_NC, nb, dim), lambda i: (0, i, 0)),
            pl.BlockSpec((_NC, nb, 16), lambda i: (0, i, 0)),
            pl.BlockSpec((nb, dim), lambda i: (i, 0)),
            _full(cb2.shape), _full(WihT.shape), _full(WhhT.shape),
            _full(bih2.shape), _full(bhh2.shape),
        ],
        out_specs=pl.BlockSpec((nb, dim), lambda i: (i, 0)),
        out_shape=jax.ShapeDtypeStruct((n, dim), jnp.float32),
    )(accp, cntp, h, cb2, WihT, WhhT, bih2, bhh2)


def _gru_update_mlp(accp, cntp, h, cb2, WihT, WhhT, bih2, bhh2,
                    g12, b12, Wl1T, bl12, Wl2T, bl22, nb):
    n, dim = h.shape
    dmid = Wl1T.shape[1]

    def body(a_ref, c_ref, h_ref, cb_ref, wi_ref, wh_ref, bi_ref, bh_ref,
             g_ref, b_ref, w1_ref, b1_ref, w2_ref, b2_ref, o_ref):
        acc = a_ref[0] + a_ref[1]
        cnt = c_ref[0][:, 0:1] + c_ref[1][:, 0:1]
        m = _rrelu(acc / jnp.maximum(cnt, 1.0) + cb_ref[...])
        hv = h_ref[...]
        gi = jnp.dot(m, wi_ref[...], preferred_element_type=jnp.float32) \
            + bi_ref[...]
        gh = jnp.dot(hv, wh_ref[...], preferred_element_type=jnp.float32) \
            + bh_ref[...]
        r = jax.nn.sigmoid(gi[:, :dim] + gh[:, :dim])
        z = jax.nn.sigmoid(gi[:, dim:2 * dim] + gh[:, dim:2 * dim])
        nn = jnp.tanh(gi[:, 2 * dim:] + r * gh[:, 2 * dim:])
        h2 = (1.0 - z) * nn + z * hv
        xb = h2 * (g_ref[...] * (1.0 / jnp.sqrt(1.0 + _EPS))) + b_ref[...]
        t = _rrelu(jnp.dot(xb, w1_ref[...],
                           preferred_element_type=jnp.float32) + b1_ref[...])
        o_ref[...] = _rrelu(jnp.dot(t, w2_ref[...],
                                    preferred_element_type=jnp.float32)
                            + b2_ref[...])

    return pl.pallas_call(
        body,
        grid=(n // nb,),
        in_specs=[
            pl.BlockSpec((_NC, nb, dim), lambda i: (0, i, 0)),
            pl.BlockSpec((_NC, nb, 16), lambda i: (0, i, 0)),
            pl.BlockSpec((nb, dim), lambda i: (i, 0)),
            _full(cb2.shape), _full(WihT.shape), _full(WhhT.shape),
            _full(bih2.shape), _full(bhh2.shape),
            _full(g12.shape), _full(b12.shape), _full(Wl1T.shape),
            _full(bl12.shape), _full(Wl2T.shape), _full(bl22.shape),
        ],
        out_specs=pl.BlockSpec((nb, dmid), lambda i: (i, 0)),
        out_shape=jax.ShapeDtypeStruct((n, dmid), jnp.float32),
    )(accp, cntp, h, cb2, WihT, WhhT, bih2, bhh2,
      g12, b12, Wl1T, bl12, Wl2T, bl22)


def _readout(rows, ea3, gN2, bN2, WlwT, Wlb2, e3, eb):
    d2 = rows.shape[1]

    def body(t0_ref, t1_ref, e_ref, g_ref, b_ref, ww_ref, wb_ref, o_ref):
        t0 = t0_ref[...]
        t1 = t1_ref[...]
        yhat = jnp.concatenate(
            [0.5 * (t0 + t1), t0 * t1, (t0 - t1) ** 2], axis=1)
        yhat = yhat * (g_ref[...] * (1.0 / jnp.sqrt(1.0 + _EPS))) + b_ref[...]
        ev = e_ref[...]
        w = jnp.dot(ev, ww_ref[...], preferred_element_type=jnp.float32)
        b = jnp.sum(ev * wb_ref[...], axis=1, keepdims=True)
        o_ref[...] = jnp.sum(yhat * w, axis=1, keepdims=True) + b

    nblk = e3 // eb
    return pl.pallas_call(
        body,
        grid=(nblk,),
        in_specs=[
            pl.BlockSpec((eb, d2), lambda i: (i, 0)),
            pl.BlockSpec((eb, d2), lambda i: (i + nblk, 0)),
            pl.BlockSpec((eb, ea3.shape[1]), lambda i: (i, 0)),
            _full(gN2.shape), _full(bN2.shape), _full(WlwT.shape),
            _full(Wlb2.shape),
        ],
        out_specs=pl.BlockSpec((eb, 1), lambda i: (i, 0)),
        out_shape=jax.ShapeDtypeStruct((e3, 1), jnp.float32),
    )(rows, rows, ea3, gN2, bN2, WlwT, Wlb2)


def _pad_to(n, mult):
    return ((n + mult - 1) // mult) * mult


def kernel(x, edge_index, edge_attr, edge_index3, edge_attr3, gx, bx, Wn,
           bn_, We, be, W1, c1b, Wih1, Whh1, bih1, bhh1, g1, b1, Wl1, bl1,
           Wl2, bl2, W2, c2b, Wih2, Whh2, bih2, bhh2, Wlw, Wlb, gN, bN):
    f32 = jnp.float32
    N = x.shape[0]
    E = edge_index.shape[1]
    E3 = edge_index3.shape[1]
    dim = Wn.shape[0]
    d2 = 2 * dim
    K1 = We.shape[0]
    K2 = edge_attr3.shape[1]
    NB = 2000
    EB = _NW * _CH

    P1 = _pad_to(E, EB)
    P2 = _pad_to(2 * E3, EB)
    nacc = _pad_to(N + 1, NB)

    ei = edge_index.astype(jnp.int32)
    ei3 = edge_index3.astype(jnp.int32)
    ch1 = P1 // (_NW * _CH)
    ch2 = P2 // (_NW * _CH)
    src1 = jnp.pad(ei[0], (0, P1 - E)).reshape(_NW, ch1, _CH)
    dst1 = jnp.pad(ei[1], (0, P1 - E),
                   constant_values=N).reshape(_NW, ch1, _CH)
    src2 = jnp.pad(jnp.concatenate([ei3[0], ei3[1]]),
                   (0, P2 - 2 * E3)).reshape(_NW, ch2, _CH)
    dst2 = jnp.pad(jnp.concatenate([ei3[1], ei3[0]]), (0, P2 - 2 * E3),
                   constant_values=N).reshape(_NW, ch2, _CH)
    ea3f = jnp.pad(jnp.concatenate([edge_attr3, edge_attr3], axis=0),
                   ((0, P2 - 2 * E3), (0, 0)))

    zeros1 = jnp.zeros((nacc, dim), f32)
    zeros2 = jnp.zeros((nacc, d2), f32)
    zerosc = jnp.zeros((nacc, 16), f32)
    ones = jnp.ones((_CH, 16), f32)

    WnT = Wn.T
    WeT = jnp.pad(We.T, ((0, 0), (0, 16 - K1)))
    be2 = jnp.pad(be, (0, 16 - K1)).reshape(1, 16)
    Vall1 = W1.reshape(dim, dim, K1).transpose(0, 2, 1).reshape(dim, K1 * dim)
    Vall2 = W2.reshape(d2, d2, K2).transpose(0, 2, 1).reshape(d2, K2 * d2)
    Bexp1 = (jnp.arange(16)[:, None] ==
             (jnp.arange(K1 * dim) // dim)[None, :]).astype(f32)
    Cred1 = ((jnp.arange(K1 * dim) % dim)[:, None] ==
             jnp.arange(dim)[None, :]).astype(f32)
    Bexp2 = (jnp.arange(K2)[:, None] ==
             (jnp.arange(K2 * d2) // d2)[None, :]).astype(f32)
    Cred2 = ((jnp.arange(K2 * d2) % d2)[:, None] ==
             jnp.arange(d2)[None, :]).astype(f32)
    gx2 = gx.reshape(1, -1)
    bx2 = bx.reshape(1, -1)
    bn2 = bn_.reshape(1, -1)
    c1b2 = c1b.reshape(1, -1)
    c2b2 = c2b.reshape(1, -1)
    Wih1T = Wih1.T
    Whh1T = Whh1.T
    bih1r = bih1.reshape(1, -1)
    bhh1r = bhh1.reshape(1, -1)
    Wih2T = Wih2.T
    Whh2T = Whh2.T
    bih2r = bih2.reshape(1, -1)
    bhh2r = bhh2.reshape(1, -1)
    g12 = g1.reshape(1, -1)
    b12 = b1.reshape(1, -1)
    Wl1T = Wl1.T
    bl12 = bl1.reshape(1, -1)
    Wl2T = Wl2.T
    bl22 = bl2.reshape(1, -1)
    gN2 = gN.reshape(1, -1)
    bN2 = bN.reshape(1, -1)
    WlwT = Wlw.T
    Wlb2 = Wlb.reshape(1, -1)

    gather1 = _build_gather(dim, P1)
    gather2 = _build_gather(d2, P2)
    scatter1 = _build_scatter(dim, P1, nacc)
    scatter2 = _build_scatter(d2, P2, nacc)
    counts = _build_counts(P1, P2, nacc)

    out0 = _node_prologue(x, gx2, bx2, WnT, bn2, NB)
    cnt1, cnt2 = counts(dst1, dst2, ones, zerosc)

    h = out0
    xj = gather1(h, src1)
    m = _edge_conv1(xj, edge_attr, WeT, be2, Bexp1, Vall1, Cred1, dim, EB)
    accp = scatter1(m, dst1, zeros1)
    h = _gru_update(accp, cnt1, h, c1b2, Wih1T, Whh1T, bih1r, bhh1r, NB)

    xj = gather1(h, src1)
    m = _edge_conv1(xj, edge_attr, WeT, be2, Bexp1, Vall1, Cred1, dim, EB)
    accp = scatter1(m, dst1, zeros1)
    h = _gru_update_mlp(accp, cnt1, h, c1b2, Wih1T, Whh1T, bih1r, bhh1r,
                        g12, b12, Wl1T, bl12, Wl2T, bl22, NB)

    for _ in range(2):
        xj = gather2(h, src2)
        m = _edge_conv2(xj, ea3f, Bexp2, Vall2, Cred2, d2, EB)
        accp = scatter2(m, dst2, zeros2)
        h = _gru_update(accp, cnt2, h, c2b2, Wih2T, Whh2T, bih2r, bhh2r, NB)

    rows = gather2(h, src2)
    res = _readout(rows, edge_attr3, gN2, bN2, WlwT, Wlb2, E3, NB)
    return res[:, 0]


# speedup vs baseline: 5.6115x; 1.7429x over previous
import functools

import jax
import jax.numpy as jnp
from jax import lax
from jax.experimental import pallas as pl
from jax.experimental.pallas import tpu as pltpu
from jax.experimental.pallas import tpu_sc as plsc

_SLOPE = (1.0 / 8.0 + 1.0 / 3.0) / 2.0
_EPS = 1e-5

_NC = 2
_NS = 16
_NW = _NC * _NS
_CH = 128


def _rrelu(v):
    return jnp.where(v >= 0, v, _SLOPE * v)


def _bd(blk, k):
    return jnp.kron(jnp.eye(k, dtype=blk.dtype), blk)


def _tile(v, k):
    return jnp.tile(v, k).reshape(1, -1)


def _sc_mesh():
    return plsc.VectorSubcoreMesh(core_axis_name="c", subcore_axis_name="s")


_SC_PARAMS = pltpu.CompilerParams(use_tc_tiling_on_sc=False)


def _build_gather(D, P):
    chunks = P // (_NW * _CH)
    per_w = chunks * _CH

    def body(table, idx3, out, idx_v, rows_v, gsem):
        wid = lax.axis_index("s") * _NC + lax.axis_index("c")
        pltpu.sync_copy(idx3.at[wid], idx_v)

        def fire(j, c):
            pltpu.async_copy(table.at[idx_v.at[j]],
                             rows_v.at[pl.ds(j * _CH, _CH)], gsem)
            return c

        lax.fori_loop(0, chunks, fire, 0)

        def drain(j, c):
            pltpu.make_async_copy(table.at[idx_v.at[0]],
                                  rows_v.at[pl.ds(0, _CH)], gsem).wait()
            return c

        lax.fori_loop(0, chunks, drain, 0)
        pltpu.sync_copy(rows_v, out.at[pl.ds(wid * per_w, per_w)])

    return pl.kernel(
        body,
        out_type=jax.ShapeDtypeStruct((P, D), jnp.float32),
        mesh=_sc_mesh(),
        scratch_types=[
            pltpu.VMEM((chunks, _CH), jnp.int32),
            pltpu.VMEM((per_w, D), jnp.float32),
            pltpu.SemaphoreType.DMA,
        ],
        compiler_params=_SC_PARAMS,
    )


def _build_scatter(D, P, nacc):
    chunks = P // (_NW * _CH)
    per_w = chunks * _CH
    rpt = nacc // _NS

    def body(rows, idx3, zeros, out, idx_v, rows_v, acc_sh, lsem, asem):
        cid = lax.axis_index("c")
        sid = lax.axis_index("s")
        wid = sid * _NC + cid
        pltpu.async_copy(idx3.at[wid], idx_v, lsem)
        pltpu.async_copy(rows.at[pl.ds(wid * per_w, per_w)], rows_v, lsem)
        pltpu.sync_copy(zeros.at[pl.ds(sid * rpt, rpt)],
                        acc_sh.at[pl.ds(sid * rpt, rpt)])
        plsc.subcore_barrier()
        pltpu.make_async_copy(idx3.at[wid], idx_v, lsem).wait()
        pltpu.make_async_copy(rows.at[pl.ds(wid * per_w, per_w)], rows_v,
                              lsem).wait()

        def fire(j, c):
            pltpu.async_copy(rows_v.at[pl.ds(j * _CH, _CH)],
                             acc_sh.at[idx_v.at[j]], asem, add=True)
            return c

        lax.fori_loop(0, chunks, fire, 0)

        def drain(j, c):
            pltpu.make_async_copy(rows_v.at[pl.ds(0, _CH)],
                                  acc_sh.at[idx_v.at[0]], asem).wait()
            return c

        lax.fori_loop(0, chunks, drain, 0)
        plsc.subcore_barrier()
        pltpu.sync_copy(acc_sh.at[pl.ds(sid * rpt, rpt)],
                        out.at[cid, pl.ds(sid * rpt, rpt)])

    return pl.kernel(
        body,
        out_type=jax.ShapeDtypeStruct((_NC, nacc, D), jnp.float32),
        mesh=_sc_mesh(),
        scratch_types=[
            pltpu.VMEM((chunks, _CH), jnp.int32),
            pltpu.VMEM((per_w, D), jnp.float32),
            pltpu.VMEM_SHARED((nacc, D), jnp.float32),
            pltpu.SemaphoreType.DMA,
            pltpu.SemaphoreType.DMA,
        ],
        compiler_params=_SC_PARAMS,
    )


def _build_counts(P1, P2, nacc):
    chunks1 = P1 // (_NW * _CH)
    chunks2 = P2 // (_NW * _CH)
    rpt = nacc // _NS

    def body(idx3a, idx3b, ones16, ones32, zeros16, zeros32, out1, out2,
             idxa_v, idxb_v, onesa_v, onesb_v, acca_sh, accb_sh,
             lsem, asem):
        cid = lax.axis_index("c")
        sid = lax.axis_index("s")
        wid = sid * _NC + cid
        pltpu.async_copy(idx3a.at[wid], idxa_v, lsem)
        pltpu.async_copy(idx3b.at[wid], idxb_v, lsem)
        pltpu.sync_copy(ones16, onesa_v)
        pltpu.sync_copy(ones32, onesb_v)
        pltpu.sync_copy(zeros16.at[pl.ds(sid * rpt, rpt)],
                        acca_sh.at[pl.ds(sid * rpt, rpt)])
        pltpu.sync_copy(zeros32.at[pl.ds(sid * rpt, rpt)],
                        accb_sh.at[pl.ds(sid * rpt, rpt)])
        plsc.subcore_barrier()
        pltpu.make_async_copy(idx3a.at[wid], idxa_v, lsem).wait()
        pltpu.make_async_copy(idx3b.at[wid], idxb_v, lsem).wait()

        def fire_a(j, c):
            pltpu.async_copy(onesa_v, acca_sh.at[idxa_v.at[j]], asem,
                             add=True)
            return c

        def fire_b(j, c):
            pltpu.async_copy(onesb_v, accb_sh.at[idxb_v.at[j]], asem,
                             add=True)
            return c

        lax.fori_loop(0, chunks1, fire_a, 0)
        lax.fori_loop(0, chunks2, fire_b, 0)

        def drain_a(j, c):
            pltpu.make_async_copy(onesa_v, acca_sh.at[idxa_v.at[0]],
                                  asem).wait()
            return c

        def drain_b(j, c):
            pltpu.make_async_copy(onesb_v, accb_sh.at[idxb_v.at[0]],
                                  asem).wait()
            return c

        lax.fori_loop(0, chunks1, drain_a, 0)
        lax.fori_loop(0, chunks2, drain_b, 0)
        plsc.subcore_barrier()
        pltpu.sync_copy(acca_sh.at[pl.ds(sid * rpt, rpt)],
                        out1.at[cid, pl.ds(sid * rpt, rpt)])
        pltpu.sync_copy(accb_sh.at[pl.ds(sid * rpt, rpt)],
                        out2.at[cid, pl.ds(sid * rpt, rpt)])

    return pl.kernel(
        body,
        out_type=[jax.ShapeDtypeStruct((_NC, nacc, 16), jnp.float32),
                  jax.ShapeDtypeStruct((_NC, nacc, 32), jnp.float32)],
        mesh=_sc_mesh(),
        scratch_types=[
            pltpu.VMEM((chunks1, _CH), jnp.int32),
            pltpu.VMEM((chunks2, _CH), jnp.int32),
            pltpu.VMEM((_CH, 16), jnp.float32),
            pltpu.VMEM((_CH, 32), jnp.float32),
            pltpu.VMEM_SHARED((nacc, 16), jnp.float32),
            pltpu.VMEM_SHARED((nacc, 32), jnp.float32),
            pltpu.SemaphoreType.DMA,
            pltpu.SemaphoreType.DMA,
        ],
        compiler_params=_SC_PARAMS,
    )


def _full(shape):
    return pl.BlockSpec(shape, lambda *_: tuple(0 for _ in shape))


def _node_prologue(x_p, gx_t, bx_t, WnBD, bn_t):
    r = x_p.shape[0]

    def body(x_ref, g_ref, b_ref, w_ref, bn_ref, o_ref):
        xb = x_ref[...] * (g_ref[...] * (1.0 / jnp.sqrt(1.0 + _EPS))) \
            + b_ref[...]
        o_ref[...] = _rrelu(
            jnp.dot(xb, w_ref[...], preferred_element_type=jnp.float32)
            + bn_ref[...])

    return pl.pallas_call(
        body,
        grid=(1,),
        in_specs=[_full(x_p.shape), _full(gx_t.shape), _full(bx_t.shape),
                  _full(WnBD.shape), _full(bn_t.shape)],
        out_specs=pl.BlockSpec((r, 256), lambda i: (0, 0)),
        out_shape=jax.ShapeDtypeStruct((r, 256), jnp.float32),
    )(x_p, gx_t, bx_t, WnBD, bn_t)


def _edge_conv1(xj_p, eattr_p, WeBD, be_t, BexpBD, VallBD, CBD, br):
    rp = xj_p.shape[0]
    ka = eattr_p.shape[1]

    def body(x_ref, e_ref, w_ref, b_ref, bx_ref, v_ref, c_ref, o_ref):
        ea = _rrelu(jnp.dot(e_ref[...], w_ref[...],
                            preferred_element_type=jnp.float32) + b_ref[...])
        eaexp = jnp.dot(ea, bx_ref[...], preferred_element_type=jnp.float32)
        proj = jnp.dot(x_ref[...], v_ref[...],
                       preferred_element_type=jnp.float32)
        o_ref[...] = jnp.dot(eaexp * proj, c_ref[...],
                             preferred_element_type=jnp.float32)

    return pl.pallas_call(
        body,
        grid=(rp // br,),
        in_specs=[
            pl.BlockSpec((br, 128), lambda i: (i, 0)),
            pl.BlockSpec((br, ka), lambda i: (i, 0)),
            _full(WeBD.shape), _full(be_t.shape), _full(BexpBD.shape),
            _full(VallBD.shape), _full(CBD.shape),
        ],
        out_specs=pl.BlockSpec((br, 128), lambda i: (i, 0)),
        out_shape=jax.ShapeDtypeStruct((rp, 128), jnp.float32),
    )(xj_p, eattr_p, WeBD, be_t, BexpBD, VallBD, CBD)


def _edge_conv2(xj_p, ea_p, BexpBD, VallBD, CBD, br):
    rp = xj_p.shape[0]
    ka = ea_p.shape[1]

    def body(x_ref, e_ref, bx_ref, v_ref, c_ref, o_ref):
        eaexp = jnp.dot(e_ref[...], bx_ref[...],
                        preferred_element_type=jnp.float32)
        proj = jnp.dot(x_ref[...], v_ref[...],
                       preferred_element_type=jnp.float32)
        o_ref[...] = jnp.dot(eaexp * proj, c_ref[...],
                             preferred_element_type=jnp.float32)

    return pl.pallas_call(
        body,
        grid=(rp // br,),
        in_specs=[
            pl.BlockSpec((br, 128), lambda i: (i, 0)),
            pl.BlockSpec((br, ka), lambda i: (i, 0)),
            _full(BexpBD.shape), _full(VallBD.shape), _full(CBD.shape),
        ],
        out_specs=pl.BlockSpec((br, 128), lambda i: (i, 0)),
        out_shape=jax.ShapeDtypeStruct((rp, 128), jnp.float32),
    )(xj_p, ea_p, BexpBD, VallBD, CBD)


def _gru_update(accp, cntp, h_p, cb_t, Wi_r, Wi_z, Wi_n, Wh_r, Wh_z, Wh_n,
                bi_t, bh_t, d, mlp=None):
    R, L = h_p.shape
    outL = 2 * L if mlp is not None else L
    if mlp is not None:
        g1_t, b1_t, Wl1BD, bl1_t, Wl2BD, bl2_t = mlp

    def body(*refs):
        if mlp is not None:
            (a_ref, c_ref, h_ref, cb_ref, wir, wiz, win, whr, whz, whn,
             bi_ref, bh_ref, g1r, b1r, w1r, b1br, w2r, b2br, o_ref) = refs
        else:
            (a_ref, c_ref, h_ref, cb_ref, wir, wiz, win, whr, whz, whn,
             bi_ref, bh_ref, o_ref) = refs
        acc = a_ref[0] + a_ref[1]
        cnt = c_ref[0] + c_ref[1]
        m = _rrelu(acc / jnp.maximum(cnt, 1.0) + cb_ref[...])
        hv = h_ref[...]

        def mm(a, w):
            return jnp.dot(a, w[...], preferred_element_type=jnp.float32)

        bi = bi_ref[...]
        bh = bh_ref[...]
        r = jax.nn.sigmoid(mm(m, wir) + bi[:, 0:L]
                           + mm(hv, whr) + bh[:, 0:L])
        z = jax.nn.sigmoid(mm(m, wiz) + bi[:, L:2 * L]
                           + mm(hv, whz) + bh[:, L:2 * L])
        nn = jnp.tanh(mm(m, win) + bi[:, 2 * L:]
                      + r * (mm(hv, whn) + bh[:, 2 * L:]))
        hnew = (1.0 - z) * nn + z * hv
        if mlp is not None:
            xb = hnew * (g1r[...] * (1.0 / jnp.sqrt(1.0 + _EPS))) + b1r[...]
            t = _rrelu(mm(xb, w1r) + b1br[...])
            o_ref[...] = _rrelu(mm(t, w2r) + b2br[...])
        else:
            o_ref[...] = hnew

    args = [accp, cntp, h_p, cb_t, Wi_r, Wi_z, Wi_n, Wh_r, Wh_z, Wh_n,
            bi_t, bh_t]
    if mlp is not None:
        args += [g1_t, b1_t, Wl1BD, bl1_t, Wl2BD, bl2_t]
    in_specs = ([pl.BlockSpec((_NC, R, L), lambda i: (0, 0, 0)),
                 pl.BlockSpec((_NC, R, L), lambda i: (0, 0, 0)),
                 _full(h_p.shape)]
                + [_full(a.shape) for a in args[3:]])
    return pl.pallas_call(
        body,
        grid=(1,),
        in_specs=in_specs,
        out_specs=pl.BlockSpec((R, outL), lambda i: (0, 0)),
        out_shape=jax.ShapeDtypeStruct((R, outL), jnp.float32),
    )(*args)


def _readout(rows_p, ea3_p, gparts, wparts, G, WlbBD, br):
    r3 = ea3_p.shape[0]
    gA, bA, gB, bB, gC, bC = gparts
    wA, wB, wC = wparts

    def body(t0_ref, t1_ref, e_ref, gar, bar, gbr, bbr, gcr, bcr,
             war, wbr, wcr, g_ref, wlb_ref, o_ref):
        t0 = t0_ref[...]
        t1 = t1_ref[...]
        s = 1.0 / jnp.sqrt(1.0 + _EPS)
        yA = (0.5 * (t0 + t1)) * (gar[...] * s) + bar[...]
        yB = (t0 * t1) * (gbr[...] * s) + bbr[...]
        yC = ((t0 - t1) ** 2) * (gcr[...] * s) + bcr[...]
        ev = e_ref[...]

        def mm(a, w):
            return jnp.dot(a, w[...], preferred_element_type=jnp.float32)

        acc = yA * mm(ev, war) + yB * mm(ev, wbr) + yC * mm(ev, wcr)
        o_ref[...] = mm(acc, g_ref) + mm(ev, wlb_ref)

    return pl.pallas_call(
        body,
        grid=(r3 // br,),
        in_specs=[
            pl.BlockSpec((br, 128), lambda i: (i, 0)),
            pl.BlockSpec((br, 128), lambda i, o=r3 // br: (i + o, 0)),
            pl.BlockSpec((br, 32), lambda i: (i, 0)),
            _full(gA.shape), _full(bA.shape), _full(gB.shape),
            _full(bB.shape), _full(gC.shape), _full(bC.shape),
            _full(wA.shape), _full(wB.shape), _full(wC.shape),
            _full(G.shape), _full(WlbBD.shape),
        ],
        out_specs=pl.BlockSpec((br, 4), lambda i: (i, 0)),
        out_shape=jax.ShapeDtypeStruct((r3, 4), jnp.float32),
    )(rows_p, rows_p, ea3_p, gA, bA, gB, bB, gC, bC, wA, wB, wC, G, WlbBD)


def _pad_to(n, mult):
    return ((n + mult - 1) // mult) * mult


def kernel(x, edge_index, edge_attr, edge_index3, edge_attr3, gx, bx, Wn,
           bn_, We, be, W1, c1b, Wih1, Whh1, bih1, bhh1, g1, b1, Wl1, bl1,
           Wl2, bl2, W2, c2b, Wih2, Whh2, bih2, bhh2, Wlw, Wlb, gN, bN):
    f32 = jnp.float32
    N = x.shape[0]
    E = edge_index.shape[1]
    E3 = edge_index3.shape[1]
    dim = Wn.shape[0]
    d2 = 2 * dim
    K1 = We.shape[0]
    K2 = edge_attr3.shape[1]
    EB = _NW * _CH

    P1 = _pad_to(E, EB)
    P2 = _pad_to(2 * E3, EB)
    NP = _pad_to(N, 128)
    nacc = _pad_to(NP + 8, 64)
    R1 = NP // 8
    RA1 = nacc * dim // 128
    RA2 = nacc * d2 // 256

    ei = edge_index.astype(jnp.int32)
    ei3 = edge_index3.astype(jnp.int32)
    ch1 = P1 // (_NW * _CH)
    ch2 = P2 // (_NW * _CH)
    src1 = jnp.pad(ei[0], (0, P1 - E)).reshape(_NW, ch1, _CH)
    dst1 = jnp.pad(ei[1], (0, P1 - E),
                   constant_values=NP).reshape(_NW, ch1, _CH)
    src2 = jnp.pad(jnp.concatenate([ei3[0], ei3[1]]),
                   (0, P2 - 2 * E3)).reshape(_NW, ch2, _CH)
    dst2 = jnp.pad(jnp.concatenate([ei3[1], ei3[0]]), (0, P2 - 2 * E3),
                   constant_values=NP).reshape(_NW, ch2, _CH)
    eattr_p = edge_attr.reshape(E // 8, 8 * edge_attr.shape[1])
    ea3f_p = jnp.pad(jnp.concatenate([edge_attr3, edge_attr3], axis=0),
                     ((0, P2 - 2 * E3), (0, 0))).reshape(P2 // 4, 4 * K2)
    ea3_p = edge_attr3.reshape(E3 // 4, 4 * K2)
    x_p = jnp.pad(x, ((0, NP - N), (0, 0))).reshape(NP // 16, 16 * x.shape[1])

    zeros1 = jnp.zeros((nacc, dim), f32)
    zeros2 = jnp.zeros((nacc, d2), f32)
    ones16 = jnp.ones((_CH, 16), f32)
    ones32 = jnp.ones((_CH, 32), f32)

    WnBD = _bd(Wn.T, 16)
    gx_t = _tile(gx, 16)
    bx_t = _tile(bx, 16)
    bn_t = _tile(bn_, 16)
    WeT = jnp.pad(We.T, ((0, 0), (0, 16 - K1)))
    WeBD = _bd(WeT, 8)
    be_t = _tile(jnp.pad(be, (0, 16 - K1)), 8)
    Vall1 = W1.reshape(dim, dim, K1).transpose(0, 2, 1).reshape(dim, K1 * dim)
    Vall2 = W2.reshape(d2, d2, K2).transpose(0, 2, 1).reshape(d2, K2 * d2)
    Bexp1 = (jnp.arange(16)[:, None] ==
             (jnp.arange(K1 * dim) // dim)[None, :]).astype(f32)
    Cred1 = ((jnp.arange(K1 * dim) % dim)[:, None] ==
             jnp.arange(dim)[None, :]).astype(f32)
    Cred2 = ((jnp.arange(K2 * d2) % d2)[:, None] ==
             jnp.arange(d2)[None, :]).astype(f32)
    BexpBD1 = _bd(Bexp1, 8)
    VallBD1 = _bd(Vall1, 8)
    CBD1 = _bd(Cred1, 8)
    Bexp2s = (jnp.arange(K2)[:, None] ==
              (jnp.arange(K2 * d2) // d2)[None, :]).astype(f32)
    BexpBD2 = _bd(Bexp2s, 4)
    VallBD2 = _bd(Vall2, 4)
    CBD2 = _bd(Cred2, 4)

    def gate_bd(W, d, k):
        WT = W.T
        return (_bd(WT[:, 0:d], k), _bd(WT[:, d:2 * d], k),
                _bd(WT[:, 2 * d:], k))

    Wi1r, Wi1z, Wi1n = gate_bd(Wih1, dim, 8)
    Wh1r, Wh1z, Wh1n = gate_bd(Whh1, dim, 8)
    bi1_t = jnp.concatenate(
        [_tile(bih1[0:dim], 8), _tile(bih1[dim:2 * dim], 8),
         _tile(bih1[2 * dim:], 8)], axis=1)
    bh1_t = jnp.concatenate(
        [_tile(bhh1[0:dim], 8), _tile(bhh1[dim:2 * dim], 8),
         _tile(bhh1[2 * dim:], 8)], axis=1)
    c1b_t = _tile(c1b, 8)
    Wi2r, Wi2z, Wi2n = gate_bd(Wih2, d2, 8)
    Wh2r, Wh2z, Wh2n = gate_bd(Whh2, d2, 8)
    bi2_t = jnp.concatenate(
        [_tile(bih2[0:d2], 8), _tile(bih2[d2:2 * d2], 8),
         _tile(bih2[2 * d2:], 8)], axis=1)
    bh2_t = jnp.concatenate(
        [_tile(bhh2[0:d2], 8), _tile(bhh2[d2:2 * d2], 8),
         _tile(bhh2[2 * d2:], 8)], axis=1)
    c2b_t = _tile(c2b, 8)
    mlp_w = (_tile(g1, 8), _tile(b1, 8), _bd(Wl1.T, 8), _tile(bl1, 8),
             _bd(Wl2.T, 8), _tile(bl2, 8))
    gA_t = _tile(gN[0:d2], 4)
    bA_t = _tile(bN[0:d2], 4)
    gB_t = _tile(gN[d2:2 * d2], 4)
    bB_t = _tile(bN[d2:2 * d2], 4)
    gC_t = _tile(gN[2 * d2:], 4)
    bC_t = _tile(bN[2 * d2:], 4)
    WlwT = Wlw.T
    wA = _bd(WlwT[:, 0:d2], 4)
    wB = _bd(WlwT[:, d2:2 * d2], 4)
    wC = _bd(WlwT[:, 2 * d2:], 4)
    Gsum = ((jnp.arange(128) // 32)[:, None] ==
            jnp.arange(4)[None, :]).astype(f32)
    WlbBD = _bd(Wlb.T, 4)

    gather1 = _build_gather(dim, P1)
    gather2 = _build_gather(d2, P2)
    scatter1 = _build_scatter(dim, P1, nacc)
    scatter2 = _build_scatter(d2, P2, nacc)
    counts = _build_counts(P1, P2, nacc)

    h_p = _node_prologue(x_p, gx_t, bx_t, WnBD, bn_t)
    h_p = h_p.reshape(R1, 128)
    cnt1, cnt2 = counts(dst1, dst2, ones16, ones32, zeros1, zeros2)
    cnt1_p = cnt1.reshape(_NC, RA1, 128)
    cnt2_p = cnt2.reshape(_NC, RA2, 256)

    for it in range(2):
        xj = gather1(h_p.reshape(NP, dim), src1)
        m_p = _edge_conv1(xj.reshape(P1 // 8, 128), eattr_p, WeBD, be_t,
                          BexpBD1, VallBD1, CBD1, 2048)
        accp = scatter1(m_p.reshape(P1, dim), dst1, zeros1)
        accp_p = accp.reshape(_NC, RA1, 128)
        mlp = mlp_w if it == 1 else None
        h_p = _gru_update(accp_p, cnt1_p, h_p, c1b_t, Wi1r, Wi1z, Wi1n,
                          Wh1r, Wh1z, Wh1n, bi1_t, bh1_t, dim, mlp=mlp)

    for it in range(2):
        xj2 = gather2(h_p.reshape(NP, d2), src2)
        m2_p = _edge_conv2(xj2.reshape(P2 // 4, 128), ea3f_p,
                           BexpBD2, VallBD2, CBD2, 2048)
        accp2 = scatter2(m2_p.reshape(P2, d2), dst2, zeros2)
        accp2_p = accp2.reshape(_NC, RA2, 256)
        h_p = _gru_update(accp2_p, cnt2_p, h_p, c2b_t, Wi2r, Wi2z, Wi2n,
                          Wh2r, Wh2z, Wh2n, bi2_t, bh2_t, d2)

    rows = gather2(h_p.reshape(NP, d2), src2)
    res = _readout(rows.reshape(P2 // 4, 128), ea3_p,
                   (gA_t, bA_t, gB_t, bB_t, gC_t, bC_t),
                   (wA, wB, wC), Gsum, WlbBD, 1000)
    return res.reshape(E3)
